# Initial kernel scaffold; baseline (speedup 1.0000x reference)
import functools

import jax
import jax.numpy as jnp
from jax import lax
from jax.experimental import pallas as pl
from jax.experimental.pallas import tpu as pltpu
from jax.experimental.pallas import tpu_sc as plsc


def _edge_block_sc(edge_attr, x, edge_index, *, chunk):
    E, DE = edge_attr.shape
    N, DF = x.shape
    DOUT = DE + 2 * DF

    info = plsc.get_sparse_core_info()
    NC, NS = info.num_cores, info.num_subcores
    NW = NC * NS
    assert E % NW == 0
    epw = E // NW
    assert epw % chunk == 0
    n_steps = epw // chunk

    mesh = plsc.VectorSubcoreMesh(core_axis_name="c", subcore_axis_name="s")

    @functools.partial(
        pl.kernel,
        mesh=mesh,
        out_type=jax.ShapeDtypeStruct((E, DOUT), jnp.float32),
        scratch_types=[
            pltpu.VMEM((chunk,), jnp.int32),
            pltpu.VMEM((chunk,), jnp.int32),
            pltpu.VMEM((chunk, DE), jnp.float32),
            pltpu.VMEM((chunk, DF), jnp.float32),
            pltpu.VMEM((chunk, DF), jnp.float32),
            pltpu.SemaphoreType.DMA,
        ],
    )
    def k(ea_hbm, x_hbm, ei_hbm, out_hbm, sidx_v, ridx_v, attr_v, rrows_v, srows_v, sem):
        wid = lax.axis_index("s") * NC + lax.axis_index("c")
        base0 = wid * epw

        def step(i, carry):
            base = base0 + i * chunk
            pltpu.sync_copy(ei_hbm.at[0, pl.ds(base, chunk)], sidx_v)
            pltpu.sync_copy(ei_hbm.at[1, pl.ds(base, chunk)], ridx_v)
            cp_r = pltpu.async_copy(x_hbm.at[ridx_v], rrows_v, sem)
            cp_s = pltpu.async_copy(x_hbm.at[sidx_v], srows_v, sem)
            pltpu.sync_copy(ea_hbm.at[pl.ds(base, chunk)], attr_v)
            pltpu.sync_copy(attr_v, out_hbm.at[pl.ds(base, chunk), pl.ds(0, DE)])
            cp_r.wait()
            pltpu.sync_copy(rrows_v, out_hbm.at[pl.ds(base, chunk), pl.ds(DE, DF)])
            cp_s.wait()
            pltpu.sync_copy(srows_v, out_hbm.at[pl.ds(base, chunk), pl.ds(DE + DF, DF)])
            return carry

        lax.fori_loop(0, n_steps, step, 0)

    return k(edge_attr, x, edge_index)


@jax.jit
def kernel(edge_attr, x, edge_index):
    return _edge_block_sc(edge_attr, x, edge_index, chunk=400)


# speedup vs baseline: 1.8638x; 1.8638x over previous
import functools

import jax
import jax.numpy as jnp
from jax import lax
from jax.experimental import pallas as pl
from jax.experimental.pallas import tpu as pltpu
from jax.experimental.pallas import tpu_sc as plsc


def _edge_block_sc(edge_attr, x, sender, receiver, *, chunk):
    E, DE = edge_attr.shape
    N, DF = x.shape
    DOUT = DE + 2 * DF

    info = plsc.get_sparse_core_info()
    NC, NS = info.num_cores, info.num_subcores
    NW = NC * NS
    assert E % NW == 0
    epw = E // NW
    assert epw % chunk == 0
    n_steps = epw // chunk

    mesh = plsc.VectorSubcoreMesh(core_axis_name="c", subcore_axis_name="s")

    @functools.partial(
        pl.kernel,
        mesh=mesh,
        compiler_params=pltpu.CompilerParams(use_tc_tiling_on_sc=False),
        out_type=jax.ShapeDtypeStruct((E, DOUT), jnp.float32),
        scratch_types=[
            pltpu.VMEM((chunk,), jnp.int32),
            pltpu.VMEM((chunk,), jnp.int32),
            pltpu.VMEM((chunk, DE), jnp.float32),
            pltpu.VMEM((chunk, DF), jnp.float32),
            pltpu.VMEM((chunk, DF), jnp.float32),
            pltpu.SemaphoreType.DMA,
        ],
    )
    def k(ea_hbm, x_hbm, snd_hbm, rcv_hbm, out_hbm, sidx_v, ridx_v, attr_v, rrows_v, srows_v, sem):
        wid = lax.axis_index("s") * NC + lax.axis_index("c")
        base0 = wid * epw

        def step(i, carry):
            base = base0 + i * chunk
            pltpu.sync_copy(snd_hbm.at[pl.ds(base, chunk)], sidx_v)
            pltpu.sync_copy(rcv_hbm.at[pl.ds(base, chunk)], ridx_v)
            cp_r = pltpu.async_copy(x_hbm.at[ridx_v], rrows_v, sem)
            cp_s = pltpu.async_copy(x_hbm.at[sidx_v], srows_v, sem)
            pltpu.sync_copy(ea_hbm.at[pl.ds(base, chunk)], attr_v)
            pltpu.sync_copy(attr_v, out_hbm.at[pl.ds(base, chunk), pl.ds(0, DE)])
            cp_r.wait()
            pltpu.sync_copy(rrows_v, out_hbm.at[pl.ds(base, chunk), pl.ds(DE, DF)])
            cp_s.wait()
            pltpu.sync_copy(srows_v, out_hbm.at[pl.ds(base, chunk), pl.ds(DE + DF, DF)])
            return carry

        lax.fori_loop(0, n_steps, step, 0)

    return k(edge_attr, x, sender, receiver)


@jax.jit
def kernel(edge_attr, x, edge_index):
    sender = edge_index[0]
    receiver = edge_index[1]
    return _edge_block_sc(edge_attr, x, sender, receiver, chunk=400)
